# Initial kernel scaffold; baseline (speedup 1.0000x reference)
import jax
import jax.numpy as jnp
from jax.experimental import pallas as pl


def kernel(data, W_word, W_pre, W_suf):
    raise NotImplementedError("write your pallas kernel here")


# speedup vs baseline: 1.4422x; 1.4422x over previous
import functools

import jax
import jax.numpy as jnp
from jax import lax
from jax.experimental import pallas as pl
from jax.experimental.pallas import tpu as pltpu
from jax.experimental.pallas import tpu_sc as plsc

_VOCAB = 1000000
_D = 32
_B = 16384
_H = 50
_N = _B * _H
_NW = 32
_G = 128
_PER_W = _N // _NW
_GROUPS = _PER_W // _G

_mesh = plsc.VectorSubcoreMesh(core_axis_name="c", subcore_axis_name="s")


@functools.partial(
    pl.kernel,
    mesh=_mesh,
    out_type=jax.ShapeDtypeStruct((_N, _D), jnp.float32),
    scratch_types=[
        pltpu.VMEM((_GROUPS, _G), jnp.int32),
        pltpu.VMEM((_G, _D), jnp.float32),
        pltpu.VMEM((_G, _D), jnp.float32),
        pltpu.VMEM((_G, _D), jnp.float32),
        pltpu.SemaphoreType.DMA,
    ],
    compiler_params=pltpu.CompilerParams(use_tc_tiling_on_sc=False),
)
def _emb_sum(idx_hbm, w_hbm, p_hbm, s_hbm, out_hbm, idx_v, b0, b1, b2, sem):
    wid = lax.axis_index("s") * 2 + lax.axis_index("c")
    pltpu.sync_copy(idx_hbm.at[pl.ds(wid * _GROUPS, _GROUPS)], idx_v)

    def group_body(g, _):
        c0 = pltpu.async_copy(w_hbm.at[idx_v.at[g]], b0, sem)
        c1 = pltpu.async_copy(p_hbm.at[idx_v.at[g]], b1, sem)
        c2 = pltpu.async_copy(s_hbm.at[idx_v.at[g]], b2, sem)
        c0.wait()
        c1.wait()
        c2.wait()

        def add_body(r, _):
            for col in (0, 16):
                sl = pl.ds(col, 16)
                b0[r, sl] = b0[r, sl] + b1[r, sl] + b2[r, sl]
            return ()

        lax.fori_loop(0, _G, add_body, (), unroll=4)
        out_off = wid * _PER_W + g * _G
        pltpu.sync_copy(b0, out_hbm.at[pl.ds(out_off, _G)])
        return ()

    lax.fori_loop(0, _GROUPS, group_body, ())


def kernel(data, W_word, W_pre, W_suf):
    idx = data.reshape(_N // _G, _G)
    out = _emb_sum(idx, W_word, W_pre, W_suf)
    return out.reshape(_B, _H, _D)


# speedup vs baseline: 1.5269x; 1.0587x over previous
import functools

import jax
import jax.numpy as jnp
from jax import lax
from jax.experimental import pallas as pl
from jax.experimental.pallas import tpu as pltpu
from jax.experimental.pallas import tpu_sc as plsc

_VOCAB = 1000000
_D = 32
_B = 16384
_H = 50
_N = _B * _H
_NW = 32
_G = 128
_PER_W = _N // _NW
_GROUPS = _PER_W // _G

_mesh = plsc.VectorSubcoreMesh(core_axis_name="c", subcore_axis_name="s")


@functools.partial(
    pl.kernel,
    mesh=_mesh,
    out_type=jax.ShapeDtypeStruct((_N, _D), jnp.float32),
    scratch_types=[
        pltpu.VMEM((_GROUPS, _G), jnp.int32),
        pltpu.VMEM((_G, _D), jnp.float32),
        pltpu.VMEM((_G, _D), jnp.float32),
        pltpu.VMEM((_G, _D), jnp.float32),
        pltpu.SemaphoreType.DMA,
    ],
    compiler_params=pltpu.CompilerParams(use_tc_tiling_on_sc=False),
)
def _emb_sum(idx_hbm, w_hbm, p_hbm, s_hbm, out_hbm, idx_v, b0, b1, b2, sem):
    wid = lax.axis_index("s") * 2 + lax.axis_index("c")
    pltpu.sync_copy(idx_hbm.at[pl.ds(wid * _GROUPS, _GROUPS)], idx_v)

    def group_body(g, _):
        c0 = pltpu.async_copy(w_hbm.at[idx_v.at[g]], b0, sem)
        c0.wait()
        c1 = pltpu.async_copy(p_hbm.at[idx_v.at[g]], b0, sem, add=True)
        c2 = pltpu.async_copy(s_hbm.at[idx_v.at[g]], b0, sem, add=True)
        c1.wait()
        c2.wait()
        out_off = wid * _PER_W + g * _G
        pltpu.sync_copy(b0, out_hbm.at[pl.ds(out_off, _G)])
        return ()

    lax.fori_loop(0, _GROUPS, group_body, ())


def kernel(data, W_word, W_pre, W_suf):
    idx = data.reshape(_N // _G, _G)
    out = _emb_sum(idx, W_word, W_pre, W_suf)
    return out.reshape(_B, _H, _D)


# speedup vs baseline: 1.6952x; 1.1102x over previous
import functools

import jax
import jax.numpy as jnp
from jax import lax
from jax.experimental import pallas as pl
from jax.experimental.pallas import tpu as pltpu
from jax.experimental.pallas import tpu_sc as plsc

_VOCAB = 1000000
_D = 32
_B = 16384
_H = 50
_N = _B * _H
_NW = 32
_G = 128
_PER_W = _N // _NW
_GROUPS = _PER_W // _G
_K = 8
_NJ = _GROUPS // _K

_mesh = plsc.VectorSubcoreMesh(core_axis_name="c", subcore_axis_name="s")


@functools.partial(
    pl.kernel,
    mesh=_mesh,
    out_type=jax.ShapeDtypeStruct((_N, _D), jnp.float32),
    scratch_types=(
        [pltpu.VMEM((_GROUPS, _G), jnp.int32)]
        + [pltpu.VMEM((_G, _D), jnp.float32) for _ in range(_K)]
        + [pltpu.SemaphoreType.DMA for _ in range(2 * _K)]
    ),
    compiler_params=pltpu.CompilerParams(use_tc_tiling_on_sc=False),
)
def _emb_sum(idx_hbm, w_hbm, p_hbm, s_hbm, out_hbm, idx_v, *scratch):
    bufs = scratch[:_K]
    gsems = scratch[_K:2 * _K]
    osems = scratch[2 * _K:]

    wid = lax.axis_index("s") * 2 + lax.axis_index("c")
    base = wid * _PER_W
    pltpu.sync_copy(idx_hbm.at[pl.ds(wid * _GROUPS, _GROUPS)], idx_v)

    def fire_w(k, g):
        return pltpu.async_copy(w_hbm.at[idx_v.at[g]], bufs[k], gsems[k])

    def fire_ps(k, g):
        pltpu.async_copy(p_hbm.at[idx_v.at[g]], bufs[k], gsems[k], add=True)
        pltpu.async_copy(s_hbm.at[idx_v.at[g]], bufs[k], gsems[k], add=True)

    def fire_out(k, g):
        return pltpu.async_copy(
            bufs[k], out_hbm.at[pl.ds(base + g * _G, _G)], osems[k])

    def drain(k, n):
        for _ in range(n):
            pltpu.make_async_copy(
                w_hbm.at[pl.ds(0, _G)], bufs[k], gsems[k]).wait()

    descs = [fire_w(k, k) for k in range(_K)]
    for k in range(_K):
        descs[k].wait()
        fire_ps(k, k)

    def body(j, _):
        outs = []
        for k in range(_K):
            drain(k, 2)
            outs.append(fire_out(k, (j - 1) * _K + k))
        wds = []
        for k in range(_K):
            outs[k].wait()
            wds.append(fire_w(k, j * _K + k))
        for k in range(_K):
            wds[k].wait()
            fire_ps(k, j * _K + k)
        return ()

    lax.fori_loop(1, _NJ, body, ())

    outs = []
    for k in range(_K):
        drain(k, 2)
        outs.append(fire_out(k, (_NJ - 1) * _K + k))
    for k in range(_K):
        outs[k].wait()


def kernel(data, W_word, W_pre, W_suf):
    idx = data.reshape(_N // _G, _G)
    out = _emb_sum(idx, W_word, W_pre, W_suf)
    return out.reshape(_B, _H, _D)


# speedup vs baseline: 1.6969x; 1.0010x over previous
import functools

import jax
import jax.numpy as jnp
from jax import lax
from jax.experimental import pallas as pl
from jax.experimental.pallas import tpu as pltpu
from jax.experimental.pallas import tpu_sc as plsc

_VOCAB = 1000000
_D = 32
_B = 16384
_H = 50
_N = _B * _H
_NW = 32
_G = 512
_PER_W = _N // _NW
_GROUPS = _PER_W // _G
_K = 5
_NJ = _GROUPS // _K

_mesh = plsc.VectorSubcoreMesh(core_axis_name="c", subcore_axis_name="s")


@functools.partial(
    pl.kernel,
    mesh=_mesh,
    out_type=jax.ShapeDtypeStruct((_N, _D), jnp.float32),
    scratch_types=(
        [pltpu.VMEM((_GROUPS, _G), jnp.int32)]
        + [pltpu.VMEM((_G, _D), jnp.float32) for _ in range(_K)]
        + [pltpu.SemaphoreType.DMA for _ in range(2 * _K)]
    ),
    compiler_params=pltpu.CompilerParams(use_tc_tiling_on_sc=False),
)
def _emb_sum(idx_hbm, w_hbm, p_hbm, s_hbm, out_hbm, idx_v, *scratch):
    bufs = scratch[:_K]
    gsems = scratch[_K:2 * _K]
    osems = scratch[2 * _K:]

    wid = lax.axis_index("s") * 2 + lax.axis_index("c")
    base = wid * _PER_W
    pltpu.sync_copy(idx_hbm.at[pl.ds(wid * _GROUPS, _GROUPS)], idx_v)

    def fire_w(k, g):
        return pltpu.async_copy(w_hbm.at[idx_v.at[g]], bufs[k], gsems[k])

    def fire_ps(k, g):
        pltpu.async_copy(p_hbm.at[idx_v.at[g]], bufs[k], gsems[k], add=True)
        pltpu.async_copy(s_hbm.at[idx_v.at[g]], bufs[k], gsems[k], add=True)

    def fire_out(k, g):
        return pltpu.async_copy(
            bufs[k], out_hbm.at[pl.ds(base + g * _G, _G)], osems[k])

    def drain(k, n):
        for _ in range(n):
            pltpu.make_async_copy(
                w_hbm.at[pl.ds(0, _G)], bufs[k], gsems[k]).wait()

    descs = [fire_w(k, k) for k in range(_K)]
    for k in range(_K):
        descs[k].wait()
        fire_ps(k, k)

    def body(j, _):
        outs = []
        for k in range(_K):
            drain(k, 2)
            outs.append(fire_out(k, (j - 1) * _K + k))
        wds = []
        for k in range(_K):
            outs[k].wait()
            wds.append(fire_w(k, j * _K + k))
        for k in range(_K):
            wds[k].wait()
            fire_ps(k, j * _K + k)
        return ()

    lax.fori_loop(1, _NJ, body, ())

    outs = []
    for k in range(_K):
        drain(k, 2)
        outs.append(fire_out(k, (_NJ - 1) * _K + k))
    for k in range(_K):
        outs[k].wait()


def kernel(data, W_word, W_pre, W_suf):
    idx = data.reshape(_N // _G, _G)
    out = _emb_sum(idx, W_word, W_pre, W_suf)
    return out.reshape(_B, _H, _D)


# speedup vs baseline: 2.2415x; 1.3210x over previous
import functools

import jax
import jax.numpy as jnp
from jax import lax
from jax.experimental import pallas as pl
from jax.experimental.pallas import tpu as pltpu
from jax.experimental.pallas import tpu_sc as plsc

_VOCAB = 1000000
_D = 32
_B = 16384
_H = 50
_NW = 32
_ROWS_W = _B // _NW
_K = 8
_NJ = _ROWS_W // _K

_mesh = plsc.VectorSubcoreMesh(core_axis_name="c", subcore_axis_name="s")


@functools.partial(
    pl.kernel,
    mesh=_mesh,
    out_type=jax.ShapeDtypeStruct((_B, _H, _D), jnp.float32),
    scratch_types=(
        [pltpu.VMEM((_ROWS_W, _H), jnp.int32)]
        + [pltpu.VMEM((_H, _D), jnp.float32) for _ in range(_K)]
        + [pltpu.SemaphoreType.DMA for _ in range(2 * _K)]
    ),
    compiler_params=pltpu.CompilerParams(use_tc_tiling_on_sc=False),
)
def _emb_sum(data_hbm, w_hbm, p_hbm, s_hbm, out_hbm, idx_v, *scratch):
    bufs = scratch[:_K]
    gsems = scratch[_K:2 * _K]
    osems = scratch[2 * _K:]

    wid = lax.axis_index("s") * 2 + lax.axis_index("c")
    base = wid * _ROWS_W
    pltpu.sync_copy(data_hbm.at[pl.ds(base, _ROWS_W)], idx_v)

    def fire_w(k, r):
        return pltpu.async_copy(w_hbm.at[idx_v.at[r]], bufs[k], gsems[k])

    def fire_ps(k, r):
        pltpu.async_copy(p_hbm.at[idx_v.at[r]], bufs[k], gsems[k], add=True)
        pltpu.async_copy(s_hbm.at[idx_v.at[r]], bufs[k], gsems[k], add=True)

    def fire_out(k, r):
        return pltpu.async_copy(bufs[k], out_hbm.at[base + r], osems[k])

    def drain(k, n):
        for _ in range(n):
            pltpu.make_async_copy(
                w_hbm.at[pl.ds(0, _H)], bufs[k], gsems[k]).wait()

    descs = [fire_w(k, k) for k in range(_K)]
    for k in range(_K):
        descs[k].wait()
        fire_ps(k, k)

    def body(j, _):
        outs = []
        for k in range(_K):
            drain(k, 2)
            outs.append(fire_out(k, (j - 1) * _K + k))
        wds = []
        for k in range(_K):
            outs[k].wait()
            wds.append(fire_w(k, j * _K + k))
        for k in range(_K):
            wds[k].wait()
            fire_ps(k, j * _K + k)
        return ()

    lax.fori_loop(1, _NJ, body, ())

    outs = []
    for k in range(_K):
        drain(k, 2)
        outs.append(fire_out(k, (_NJ - 1) * _K + k))
    for k in range(_K):
        outs[k].wait()


def kernel(data, W_word, W_pre, W_suf):
    return _emb_sum(data, W_word, W_pre, W_suf)
